# baseline (speedup 1.0000x reference)
import jax
import jax.numpy as jnp
from jax.experimental import pallas as pl
from jax.experimental.pallas import tpu as pltpu

TAU = 0.07
EPS = 1e-12

B = 4
C = 512
K = 21
HW = 64 * 64
CCH = 64
NJ = C // CCH
T_PX = HW // NJ


def _body(feat_ref, gt_ref, out_ref, xb_scr, s2_scr, rn_scr, k0_scr, k0n_scr,
          acc_ref):
    phase = pl.program_id(0)
    b = pl.program_id(1)
    j = pl.program_id(2)

    @pl.when((phase == 0) & (b == 0) & (j == 0))
    def _init():
        k0_scr[...] = jnp.zeros_like(k0_scr)
        acc_ref[0] = 0.0
        acc_ref[1] = 0.0

    @pl.when(phase == 0)
    def _phase0():
        x = feat_ref[0]
        xb = x.astype(jnp.bfloat16)
        xb_scr[pl.ds(b, 1), pl.ds(j * CCH, CCH)] = xb[None]
        ones = jnp.ones((8, CCH), jnp.bfloat16)
        part = jax.lax.dot_general(
            ones, xb * xb,
            dimension_numbers=(((1,), (0,)), ((), ())),
            preferred_element_type=jnp.float32)

        @pl.when(j == 0)
        def _s2_set():
            s2_scr[...] = part

        @pl.when(j > 0)
        def _s2_acc():
            s2_scr[...] += part

        @pl.when(j == NJ - 1)
        def _batch_done():
            s2 = s2_scr[0:1]
            rnorm = 1.0 / jnp.maximum(jnp.sqrt(s2), EPS)
            rn_scr[pl.ds(b, 1)] = rnorm[None]
            maskf = (gt_ref[0] == 1).astype(jnp.float32)
            wmask = (maskf * rnorm).astype(jnp.bfloat16)
            k0_scr[...] += jax.lax.dot_general(
                wmask, xb_scr[b],
                dimension_numbers=(((1,), (1,)), ((), ())),
                preferred_element_type=jnp.float32)

        @pl.when((b == B - 1) & (j == NJ - 1))
        def _finalize_k0():
            k0 = k0_scr[...]
            nrm = jnp.sqrt(jnp.sum(k0 * k0, axis=1, keepdims=True))
            k0n_scr[...] = (k0 / jnp.maximum(nrm, EPS)).astype(jnp.bfloat16)

    @pl.when(phase == 1)
    def _phase1():
        xb_t = xb_scr[b, :, pl.ds(j * T_PX, T_PX)]
        simraw = jax.lax.dot_general(
            k0n_scr[...], xb_t,
            dimension_numbers=(((1,), (0,)), ((), ())),
            preferred_element_type=jnp.float32)
        rnorm = rn_scr[b, :, pl.ds(j * T_PX, T_PX)]
        sim = simraw * (rnorm * (1.0 / TAU))
        maskf = (gt_ref[0, :, pl.ds(j * T_PX, T_PX)] == 1).astype(jnp.float32)
        mx = jnp.max(sim, axis=0, keepdims=True)
        lse = mx + jnp.log(jnp.sum(jnp.exp(sim - mx), axis=0, keepdims=True))
        m = jnp.sum(maskf, axis=0, keepdims=True)
        acc_ref[0] += jnp.sum(maskf)
        acc_ref[1] += jnp.sum(maskf * sim) - jnp.sum(m * lse)

        @pl.when((b == B - 1) & (j == NJ - 1))
        def _final():
            out_ref[...] = jnp.broadcast_to(-acc_ref[1] / acc_ref[0], (1, 1))


def kernel(feat, gt):
    feat2 = feat.reshape(B, C, HW)
    gt2 = gt.reshape(B, K, HW)
    out = pl.pallas_call(
        _body,
        grid=(2, B, NJ),
        in_specs=[
            pl.BlockSpec(
                (1, CCH, HW),
                lambda p, b, j: (jnp.where(p == 0, b, B - 1),
                                 jnp.where(p == 0, j, NJ - 1), 0)),
            pl.BlockSpec((1, K, HW), lambda p, b, j: (b, 0, 0)),
        ],
        out_specs=pl.BlockSpec((1, 1), lambda p, b, j: (0, 0)),
        out_shape=jax.ShapeDtypeStruct((1, 1), jnp.float32),
        scratch_shapes=[
            pltpu.VMEM((B, C, HW), jnp.bfloat16),
            pltpu.VMEM((8, HW), jnp.float32),
            pltpu.VMEM((B, 1, HW), jnp.float32),
            pltpu.VMEM((K, C), jnp.float32),
            pltpu.VMEM((K, C), jnp.bfloat16),
            pltpu.SMEM((2,), jnp.float32),
        ],
    )(feat2, gt2)
    return out.reshape(1)
